# baseline (speedup 1.0000x reference)
import functools
import jax
import jax.numpy as jnp
from jax import lax
from jax.experimental import pallas as pl
from jax.experimental.pallas import tpu as pltpu
from jax.experimental.pallas import tpu_sc as plsc

N_NODES = 50000
N_EDGES = 800000
DIM = 64
HD = 32
NGAUSS = 64
NCONV = 3
CUTOFF = 5.0
NTYPES = 100
LOG2 = 0.6931471805599453

NSUB = 16
CHUNK = 80
EPT = 51200
E_PAD = EPT * NSUB
NPT = 3136
N_PAD = NPT * NSUB
GARBAGE_ROW = N_NODES

BN = 1568
BE = 2048


def _ssp(x):
    return jnp.maximum(x, 0.0) + jnp.log1p(jnp.exp(-jnp.abs(x))) - LOG2


def _dotd(a, b):
    return jnp.dot(a.astype(jnp.bfloat16), b.astype(jnp.bfloat16),
                   preferred_element_type=jnp.float32)


def _emb_nh_body(nt_ref, emb_ref, wn_ref, bn_ref, h_ref, nh0_ref, nh1_ref):
    ids = nt_ref[...]
    cols = lax.broadcasted_iota(jnp.int32, (BN, 128), 1)
    oh = (ids == cols).astype(jnp.float32)
    h = jnp.dot(oh, emb_ref[...], preferred_element_type=jnp.float32,
                precision=jax.lax.Precision.HIGHEST)
    nh = _dotd(h, wn_ref[...]) + bn_ref[...]
    h_ref[...] = h
    nh0_ref[...] = nh[:, :HD]
    nh1_ref[...] = nh[:, HD:]


def _emb_nh(nt_p, emb_pad, wn, bn):
    grid = N_PAD // BN
    return pl.pallas_call(
        _emb_nh_body,
        grid=(grid,),
        in_specs=[
            pl.BlockSpec((BN, 1), lambda i: (i, 0)),
            pl.BlockSpec((128, DIM), lambda i: (0, 0)),
            pl.BlockSpec((DIM, DIM), lambda i: (0, 0)),
            pl.BlockSpec((1, DIM), lambda i: (0, 0)),
        ],
        out_specs=[
            pl.BlockSpec((BN, DIM), lambda i: (i, 0)),
            pl.BlockSpec((BN, HD), lambda i: (i, 0)),
            pl.BlockSpec((BN, HD), lambda i: (i, 0)),
        ],
        out_shape=[
            jax.ShapeDtypeStruct((N_PAD, DIM), jnp.float32),
            jax.ShapeDtypeStruct((N_PAD, HD), jnp.float32),
            jax.ShapeDtypeStruct((N_PAD, HD), jnp.float32),
        ],
    )(nt_p, emb_pad, wn, bn)


def _efilter_body(dist_ref, offs_ref, coeff_ref, w1_ref, b1_ref, w2_ref, b2_ref,
                  *e_refs):
    dt = dist_ref[0]
    offs = offs_ref[...]
    coeff = coeff_ref[...]
    slabs = []
    for r in range(BE // 128):
        dcol = dt[:, r:r + 1]
        d = dcol - offs
        slabs.append(jnp.exp(coeff * d * d))
    rbf = jnp.concatenate(slabs, axis=0)
    x = _ssp(_dotd(rbf, w1_ref[...]) + b1_ref[...])
    e = _ssp(_dotd(x, w2_ref[...]) + b2_ref[...])
    e_refs[0][...] = e[:, :HD]
    e_refs[1][...] = e[:, HD:]


def _efilter1(dist_t, offs, coeff, w1, b1, w2, b2):
    grid = E_PAD // BE
    return pl.pallas_call(
        _efilter_body,
        grid=(grid,),
        in_specs=[
            pl.BlockSpec((1, 128, BE // 128), lambda i: (i, 0, 0)),
            pl.BlockSpec((1, NGAUSS), lambda i: (0, 0)),
            pl.BlockSpec((1, 1), lambda i: (0, 0)),
            pl.BlockSpec((NGAUSS, DIM), lambda i: (0, 0)),
            pl.BlockSpec((1, DIM), lambda i: (0, 0)),
            pl.BlockSpec((DIM, DIM), lambda i: (0, 0)),
            pl.BlockSpec((1, DIM), lambda i: (0, 0)),
        ],
        out_specs=[pl.BlockSpec((BE, HD), lambda i: (i, 0))] * 2,
        out_shape=[jax.ShapeDtypeStruct((E_PAD, HD), jnp.float32)] * 2,
    )(dist_t, offs, coeff, w1, b1, w2, b2)


class _SemList:
    def __init__(self, refs):
        self.at = list(refs)


NZB = 98
NBUF = 5
NGROUP = EPT // CHUNK // NBUF


def _sc_conv_body(nh0, nh1, e0, e1, srcI, dstI, agg0, agg1,
                  src_i, dst_i, e_b, g_b, zb, aggsh, *sems):
    sem_in = _SemList(sems[0:NBUF])
    sem_g = _SemList(sems[NBUF:2 * NBUF])
    sem_s = _SemList(sems[2 * NBUF:3 * NBUF])
    cid = lax.axis_index("c")
    sid = lax.axis_index("s")

    zeros = jnp.zeros((16,), jnp.float32)

    def zrow(r, _):
        zb[r, pl.ds(0, 16)] = zeros
        zb[r, pl.ds(16, 16)] = zeros
        return 0
    lax.fori_loop(0, NZB, zrow, 0)
    for r in range(NPT // NZB):
        pltpu.sync_copy(zb, aggsh.at[pl.ds(sid * NPT + r * NZB, NZB)])
    plsc.subcore_barrier()

    def run_half(nh_h, e_h, agg_out):
        def group(go, _):
            j0 = go * NBUF

            @pl.when(go > 0)
            def _():
                for b in range(NBUF):
                    pltpu.make_async_copy(
                        g_b.at[b], aggsh.at[dst_i.at[b]], sem_s.at[b]).wait()

            for b in range(NBUF):
                base = sid * EPT + (j0 + b) * CHUNK
                pltpu.async_copy(srcI.at[pl.ds(base, CHUNK)], src_i.at[b],
                                 sem_in.at[b])
                pltpu.async_copy(dstI.at[pl.ds(base, CHUNK)], dst_i.at[b],
                                 sem_in.at[b])
                pltpu.async_copy(e_h.at[pl.ds(base, CHUNK)], e_b.at[b],
                                 sem_in.at[b])

            for b in range(NBUF):
                base = sid * EPT + (j0 + b) * CHUNK
                pltpu.make_async_copy(srcI.at[pl.ds(base, CHUNK)], src_i.at[b],
                                      sem_in.at[b]).wait()
                pltpu.make_async_copy(dstI.at[pl.ds(base, CHUNK)], dst_i.at[b],
                                      sem_in.at[b]).wait()
                pltpu.make_async_copy(e_h.at[pl.ds(base, CHUNK)], e_b.at[b],
                                      sem_in.at[b]).wait()
                pltpu.async_copy(nh_h.at[src_i.at[b]], g_b.at[b], sem_g.at[b])

            for b in range(NBUF):
                pltpu.make_async_copy(nh_h.at[src_i.at[b]], g_b.at[b],
                                      sem_g.at[b]).wait()

                def rows(i, _, b=b):
                    for rr in range(8):
                        r = i * 8 + rr
                        for c2 in range(2):
                            sl = pl.ds(c2 * 16, 16)
                            g_b[b, r, sl] = g_b[b, r, sl] * e_b[b, r, sl]
                    return 0
                lax.fori_loop(0, CHUNK // 8, rows, 0)
                pltpu.async_copy(g_b.at[b], aggsh.at[dst_i.at[b]], sem_s.at[b],
                                 add=True)
            return 0
        lax.fori_loop(0, NGROUP, group, 0)
        for b in range(NBUF):
            pltpu.make_async_copy(g_b.at[b], aggsh.at[dst_i.at[b]],
                                  sem_s.at[b]).wait()
        plsc.subcore_barrier()
        pltpu.sync_copy(aggsh.at[pl.ds(sid * NPT, NPT)],
                        agg_out.at[pl.ds(sid * NPT, NPT)])

    @pl.when(cid == 0)
    def _():
        run_half(nh0, e0, agg0)

    @pl.when(cid == 1)
    def _():
        run_half(nh1, e1, agg1)


@functools.cache
def _make_sc_conv():
    return functools.partial(
        pl.kernel,
        out_type=[
            jax.ShapeDtypeStruct((N_PAD, HD), jnp.float32),
            jax.ShapeDtypeStruct((N_PAD, HD), jnp.float32),
        ],
        mesh=plsc.VectorSubcoreMesh(core_axis_name="c", subcore_axis_name="s",
                                    num_cores=2, num_subcores=NSUB),
        scratch_types=[
            pltpu.VMEM((NBUF, CHUNK), jnp.int32),
            pltpu.VMEM((NBUF, CHUNK), jnp.int32),
            pltpu.VMEM((NBUF, CHUNK, HD), jnp.float32),
            pltpu.VMEM((NBUF, CHUNK, HD), jnp.float32),
            pltpu.VMEM((NZB, HD), jnp.float32),
            pltpu.VMEM_SHARED((N_PAD, HD), jnp.float32),
        ] + [pltpu.SemaphoreType.DMA] * (3 * NBUF) + [
        ],
        compiler_params=pltpu.CompilerParams(use_tc_tiling_on_sc=False),
    )(_sc_conv_body)


def _sc_conv(*args):
    return _make_sc_conv()(*args)


def _mid_body(a0_ref, a1_ref, h_ref, wc_ref, bc_ref, wo_ref, bo_ref,
              wn_ref, bn_ref, hn_ref, nh0_ref, nh1_ref):
    agg = jnp.concatenate([a0_ref[...], a1_ref[...]], axis=1)
    agg = _ssp(_dotd(agg, wc_ref[...]) + bc_ref[...])
    agg = _dotd(agg, wo_ref[...]) + bo_ref[...]
    hn = h_ref[...] + agg
    nh = _dotd(hn, wn_ref[...]) + bn_ref[...]
    hn_ref[...] = hn
    nh0_ref[...] = nh[:, :HD]
    nh1_ref[...] = nh[:, HD:]


def _mid(a0, a1, h, wc, bc, wo, bo, wn, bn):
    grid = N_PAD // BN
    wspec = pl.BlockSpec((DIM, DIM), lambda i: (0, 0))
    bspec = pl.BlockSpec((1, DIM), lambda i: (0, 0))
    return pl.pallas_call(
        _mid_body,
        grid=(grid,),
        in_specs=[
            pl.BlockSpec((BN, HD), lambda i: (i, 0)),
            pl.BlockSpec((BN, HD), lambda i: (i, 0)),
            pl.BlockSpec((BN, DIM), lambda i: (i, 0)),
            wspec, bspec, wspec, bspec, wspec, bspec,
        ],
        out_specs=[
            pl.BlockSpec((BN, DIM), lambda i: (i, 0)),
            pl.BlockSpec((BN, HD), lambda i: (i, 0)),
            pl.BlockSpec((BN, HD), lambda i: (i, 0)),
        ],
        out_shape=[
            jax.ShapeDtypeStruct((N_PAD, DIM), jnp.float32),
            jax.ShapeDtypeStruct((N_PAD, HD), jnp.float32),
            jax.ShapeDtypeStruct((N_PAD, HD), jnp.float32),
        ],
    )(a0, a1, h, wc, bc, wo, bo, wn, bn)


def _final_body(a0_ref, a1_ref, h_ref, wc_ref, bc_ref, wo_ref, bo_ref,
                wr1_ref, br1_ref, wr2_ref, br2_ref, out_ref):
    i = pl.program_id(0)
    agg = jnp.concatenate([a0_ref[...], a1_ref[...]], axis=1)
    agg = _ssp(_dotd(agg, wc_ref[...]) + bc_ref[...])
    agg = _dotd(agg, wo_ref[...]) + bo_ref[...]
    hn = h_ref[...] + agg
    o = _ssp(_dotd(hn, wr1_ref[...]) + br1_ref[...])
    o = _dotd(o, wr2_ref[...]) + br2_ref[...]
    rows = i * BN + lax.broadcasted_iota(jnp.int32, (BN, 1), 0)
    o = jnp.where(rows < N_NODES, o, 0.0)

    @pl.when(i == 0)
    def _():
        out_ref[...] = jnp.zeros_like(out_ref)
    out_ref[...] += jnp.sum(o, axis=(0, 1), keepdims=True) * (1.0 / N_NODES)


def _final(a0, a1, h, wc, bc, wo, bo, wr1, br1, wr2, br2):
    grid = N_PAD // BN
    wspec = pl.BlockSpec((DIM, DIM), lambda i: (0, 0))
    bspec = pl.BlockSpec((1, DIM), lambda i: (0, 0))
    return pl.pallas_call(
        _final_body,
        grid=(grid,),
        in_specs=[
            pl.BlockSpec((BN, HD), lambda i: (i, 0)),
            pl.BlockSpec((BN, HD), lambda i: (i, 0)),
            pl.BlockSpec((BN, DIM), lambda i: (i, 0)),
            wspec, bspec, wspec, bspec,
            wspec, bspec,
            pl.BlockSpec((DIM, 1), lambda i: (0, 0)),
            pl.BlockSpec((1, 1), lambda i: (0, 0)),
        ],
        out_specs=pl.BlockSpec((1, 1), lambda i: (0, 0)),
        out_shape=jax.ShapeDtypeStruct((1, 1), jnp.float32),
    )(a0, a1, h, wc, bc, wo, bo, wr1, br1, wr2, br2)


def kernel(node_type, edge_index, dist, emb_table, Wn, bn, We1, be1, We2, be2,
           Wc, bc, Wo, bo, Wr1, br1, Wr2, br2):
    src = jnp.pad(edge_index[0].astype(jnp.int32), (0, E_PAD - N_EDGES))
    dst = jnp.pad(edge_index[1].astype(jnp.int32), (0, E_PAD - N_EDGES),
                  constant_values=GARBAGE_ROW)
    dist_t = (jnp.pad(dist, (0, E_PAD - N_EDGES))
              .reshape(E_PAD // BE, BE // 128, 128).transpose(0, 2, 1))
    nt_p = jnp.pad(node_type.astype(jnp.int32), (0, N_PAD - N_NODES))[:, None]
    emb_pad = jnp.pad(emb_table, ((0, 128 - NTYPES), (0, 0)))

    offset = jnp.linspace(0.0, CUTOFF, NGAUSS)
    coeff = (-0.5 / ((offset[1] - offset[0]) ** 2)).reshape(1, 1)
    offs = offset.reshape(1, NGAUSS)

    e_halves = [
        _efilter1(dist_t, offs, coeff, We1[i], be1[i][None], We2[i],
                  be2[i][None])
        for i in range(NCONV)
    ]
    h, nh0, nh1 = _emb_nh(nt_p, emb_pad, Wn[0], bn[0][None])
    for i in range(NCONV):
        a0, a1 = _sc_conv(nh0, nh1, e_halves[i][0], e_halves[i][1], src, dst)
        if i + 1 < NCONV:
            h, nh0, nh1 = _mid(a0, a1, h, Wc[i], bc[i][None], Wo[i], bo[i][None],
                               Wn[i + 1], bn[i + 1][None])
        else:
            tot = _final(a0, a1, h, Wc[i], bc[i][None], Wo[i], bo[i][None],
                         Wr1, br1[None], Wr2, br2[None])
    return tot.reshape(1)


# speedup vs baseline: 1.0215x; 1.0215x over previous
import functools
import jax
import jax.numpy as jnp
from jax import lax
from jax.experimental import pallas as pl
from jax.experimental.pallas import tpu as pltpu
from jax.experimental.pallas import tpu_sc as plsc

N_NODES = 50000
N_EDGES = 800000
DIM = 64
HD = 32
NGAUSS = 64
NCONV = 3
CUTOFF = 5.0
NTYPES = 100
LOG2 = 0.6931471805599453

NSUB = 16
CHUNK = 40
EPT = 51200
E_PAD = EPT * NSUB
NPT = 3136
N_PAD = NPT * NSUB
GARBAGE_ROW = N_NODES

BN = 1568
BE = 2048


def _ssp(x):
    return jnp.maximum(x, 0.0) + jnp.log1p(jnp.exp(-jnp.abs(x))) - LOG2


def _dotd(a, b):
    return jnp.dot(a.astype(jnp.bfloat16), b.astype(jnp.bfloat16),
                   preferred_element_type=jnp.float32)


def _emb_nh_body(nt_ref, emb_ref, wn_ref, bn_ref, h_ref, nh0_ref, nh1_ref):
    ids = nt_ref[...]
    cols = lax.broadcasted_iota(jnp.int32, (BN, 128), 1)
    oh = (ids == cols).astype(jnp.float32)
    h = jnp.dot(oh, emb_ref[...], preferred_element_type=jnp.float32,
                precision=jax.lax.Precision.HIGHEST)
    nh = _dotd(h, wn_ref[...]) + bn_ref[...]
    h_ref[...] = h
    nh0_ref[...] = nh[:, :HD]
    nh1_ref[...] = nh[:, HD:]


def _emb_nh(nt_p, emb_pad, wn, bn):
    grid = N_PAD // BN
    return pl.pallas_call(
        _emb_nh_body,
        grid=(grid,),
        in_specs=[
            pl.BlockSpec((BN, 1), lambda i: (i, 0)),
            pl.BlockSpec((128, DIM), lambda i: (0, 0)),
            pl.BlockSpec((DIM, DIM), lambda i: (0, 0)),
            pl.BlockSpec((1, DIM), lambda i: (0, 0)),
        ],
        out_specs=[
            pl.BlockSpec((BN, DIM), lambda i: (i, 0)),
            pl.BlockSpec((BN, HD), lambda i: (i, 0)),
            pl.BlockSpec((BN, HD), lambda i: (i, 0)),
        ],
        out_shape=[
            jax.ShapeDtypeStruct((N_PAD, DIM), jnp.float32),
            jax.ShapeDtypeStruct((N_PAD, HD), jnp.float32),
            jax.ShapeDtypeStruct((N_PAD, HD), jnp.float32),
        ],
    )(nt_p, emb_pad, wn, bn)


def _efilter_body(dist_ref, offs_ref, coeff_ref, w1_ref, b1_ref, w2_ref, b2_ref,
                  *e_refs):
    dt = dist_ref[0]
    offs = offs_ref[...]
    coeff = coeff_ref[...]
    slabs = []
    for r in range(BE // 128):
        dcol = dt[:, r:r + 1]
        d = dcol - offs
        slabs.append(jnp.exp(coeff * d * d))
    rbf = jnp.concatenate(slabs, axis=0)
    for i in range(NCONV):
        x = _ssp(_dotd(rbf, w1_ref[i]) + b1_ref[i])
        e = _ssp(_dotd(x, w2_ref[i]) + b2_ref[i])
        e_refs[2 * i][...] = e[:, :HD]
        e_refs[2 * i + 1][...] = e[:, HD:]


def _efilter3(dist_t, offs, coeff, w1s, b1s, w2s, b2s):
    grid = E_PAD // BE
    return pl.pallas_call(
        _efilter_body,
        grid=(grid,),
        in_specs=[
            pl.BlockSpec((1, 128, BE // 128), lambda i: (i, 0, 0)),
            pl.BlockSpec((1, NGAUSS), lambda i: (0, 0)),
            pl.BlockSpec((1, 1), lambda i: (0, 0)),
            pl.BlockSpec((NCONV, NGAUSS, DIM), lambda i: (0, 0, 0)),
            pl.BlockSpec((NCONV, 1, DIM), lambda i: (0, 0, 0)),
            pl.BlockSpec((NCONV, DIM, DIM), lambda i: (0, 0, 0)),
            pl.BlockSpec((NCONV, 1, DIM), lambda i: (0, 0, 0)),
        ],
        out_specs=[pl.BlockSpec((BE, HD), lambda i: (i, 0))] * (2 * NCONV),
        out_shape=[jax.ShapeDtypeStruct((E_PAD, HD), jnp.float32)] * (2 * NCONV),
    )(dist_t, offs, coeff, w1s, b1s, w2s, b2s)


class _SemList:
    def __init__(self, refs):
        self.at = list(refs)


NZB = 98
NBUF = 10
NGROUP = EPT // CHUNK // NBUF


def _sc_conv_body(nh0, nh1, e0, e1, srcI, dstI, agg0, agg1,
                  src_i, dst_i, e_b, g_b, zb, aggsh, *sems):
    sem_in = _SemList(sems[0:NBUF])
    sem_g = _SemList(sems[NBUF:2 * NBUF])
    sem_s = _SemList(sems[2 * NBUF:3 * NBUF])
    cid = lax.axis_index("c")
    sid = lax.axis_index("s")

    zeros = jnp.zeros((16,), jnp.float32)

    def zrow(r, _):
        zb[r, pl.ds(0, 16)] = zeros
        zb[r, pl.ds(16, 16)] = zeros
        return 0
    lax.fori_loop(0, NZB, zrow, 0)
    for r in range(NPT // NZB):
        pltpu.sync_copy(zb, aggsh.at[pl.ds(sid * NPT + r * NZB, NZB)])
    plsc.subcore_barrier()

    def run_half(nh_h, e_h, agg_out):
        def group(go, _):
            j0 = go * NBUF

            @pl.when(go > 0)
            def _():
                for b in range(NBUF):
                    pltpu.make_async_copy(
                        g_b.at[b], aggsh.at[dst_i.at[b]], sem_s.at[b]).wait()

            for b in range(NBUF):
                base = sid * EPT + (j0 + b) * CHUNK
                pltpu.async_copy(srcI.at[pl.ds(base, CHUNK)], src_i.at[b],
                                 sem_in.at[b])
                pltpu.async_copy(dstI.at[pl.ds(base, CHUNK)], dst_i.at[b],
                                 sem_in.at[b])
                pltpu.async_copy(e_h.at[pl.ds(base, CHUNK)], e_b.at[b],
                                 sem_in.at[b])

            for b in range(NBUF):
                base = sid * EPT + (j0 + b) * CHUNK
                pltpu.make_async_copy(srcI.at[pl.ds(base, CHUNK)], src_i.at[b],
                                      sem_in.at[b]).wait()
                pltpu.make_async_copy(dstI.at[pl.ds(base, CHUNK)], dst_i.at[b],
                                      sem_in.at[b]).wait()
                pltpu.make_async_copy(e_h.at[pl.ds(base, CHUNK)], e_b.at[b],
                                      sem_in.at[b]).wait()
                pltpu.async_copy(nh_h.at[src_i.at[b]], g_b.at[b], sem_g.at[b])

            for b in range(NBUF):
                pltpu.make_async_copy(nh_h.at[src_i.at[b]], g_b.at[b],
                                      sem_g.at[b]).wait()

                def rows(i, _, b=b):
                    for rr in range(8):
                        r = i * 8 + rr
                        for c2 in range(2):
                            sl = pl.ds(c2 * 16, 16)
                            g_b[b, r, sl] = g_b[b, r, sl] * e_b[b, r, sl]
                    return 0
                lax.fori_loop(0, CHUNK // 8, rows, 0)
                pltpu.async_copy(g_b.at[b], aggsh.at[dst_i.at[b]], sem_s.at[b],
                                 add=True)
            return 0
        lax.fori_loop(0, NGROUP, group, 0)
        for b in range(NBUF):
            pltpu.make_async_copy(g_b.at[b], aggsh.at[dst_i.at[b]],
                                  sem_s.at[b]).wait()
        plsc.subcore_barrier()
        pltpu.sync_copy(aggsh.at[pl.ds(sid * NPT, NPT)],
                        agg_out.at[pl.ds(sid * NPT, NPT)])

    @pl.when(cid == 0)
    def _():
        run_half(nh0, e0, agg0)

    @pl.when(cid == 1)
    def _():
        run_half(nh1, e1, agg1)


@functools.cache
def _make_sc_conv():
    return functools.partial(
        pl.kernel,
        out_type=[
            jax.ShapeDtypeStruct((N_PAD, HD), jnp.float32),
            jax.ShapeDtypeStruct((N_PAD, HD), jnp.float32),
        ],
        mesh=plsc.VectorSubcoreMesh(core_axis_name="c", subcore_axis_name="s",
                                    num_cores=2, num_subcores=NSUB),
        scratch_types=[
            pltpu.VMEM((NBUF, CHUNK), jnp.int32),
            pltpu.VMEM((NBUF, CHUNK), jnp.int32),
            pltpu.VMEM((NBUF, CHUNK, HD), jnp.float32),
            pltpu.VMEM((NBUF, CHUNK, HD), jnp.float32),
            pltpu.VMEM((NZB, HD), jnp.float32),
            pltpu.VMEM_SHARED((N_PAD, HD), jnp.float32),
        ] + [pltpu.SemaphoreType.DMA] * (3 * NBUF) + [
        ],
        compiler_params=pltpu.CompilerParams(use_tc_tiling_on_sc=False),
    )(_sc_conv_body)


def _sc_conv(*args):
    return _make_sc_conv()(*args)


def _mid_body(a0_ref, a1_ref, h_ref, wc_ref, bc_ref, wo_ref, bo_ref,
              wn_ref, bn_ref, hn_ref, nh0_ref, nh1_ref):
    agg = jnp.concatenate([a0_ref[...], a1_ref[...]], axis=1)
    agg = _ssp(_dotd(agg, wc_ref[...]) + bc_ref[...])
    agg = _dotd(agg, wo_ref[...]) + bo_ref[...]
    hn = h_ref[...] + agg
    nh = _dotd(hn, wn_ref[...]) + bn_ref[...]
    hn_ref[...] = hn
    nh0_ref[...] = nh[:, :HD]
    nh1_ref[...] = nh[:, HD:]


def _mid(a0, a1, h, wc, bc, wo, bo, wn, bn):
    grid = N_PAD // BN
    wspec = pl.BlockSpec((DIM, DIM), lambda i: (0, 0))
    bspec = pl.BlockSpec((1, DIM), lambda i: (0, 0))
    return pl.pallas_call(
        _mid_body,
        grid=(grid,),
        in_specs=[
            pl.BlockSpec((BN, HD), lambda i: (i, 0)),
            pl.BlockSpec((BN, HD), lambda i: (i, 0)),
            pl.BlockSpec((BN, DIM), lambda i: (i, 0)),
            wspec, bspec, wspec, bspec, wspec, bspec,
        ],
        out_specs=[
            pl.BlockSpec((BN, DIM), lambda i: (i, 0)),
            pl.BlockSpec((BN, HD), lambda i: (i, 0)),
            pl.BlockSpec((BN, HD), lambda i: (i, 0)),
        ],
        out_shape=[
            jax.ShapeDtypeStruct((N_PAD, DIM), jnp.float32),
            jax.ShapeDtypeStruct((N_PAD, HD), jnp.float32),
            jax.ShapeDtypeStruct((N_PAD, HD), jnp.float32),
        ],
    )(a0, a1, h, wc, bc, wo, bo, wn, bn)


def _final_body(a0_ref, a1_ref, h_ref, wc_ref, bc_ref, wo_ref, bo_ref,
                wr1_ref, br1_ref, wr2_ref, br2_ref, out_ref):
    i = pl.program_id(0)
    agg = jnp.concatenate([a0_ref[...], a1_ref[...]], axis=1)
    agg = _ssp(_dotd(agg, wc_ref[...]) + bc_ref[...])
    agg = _dotd(agg, wo_ref[...]) + bo_ref[...]
    hn = h_ref[...] + agg
    o = _ssp(_dotd(hn, wr1_ref[...]) + br1_ref[...])
    o = _dotd(o, wr2_ref[...]) + br2_ref[...]
    rows = i * BN + lax.broadcasted_iota(jnp.int32, (BN, 1), 0)
    o = jnp.where(rows < N_NODES, o, 0.0)

    @pl.when(i == 0)
    def _():
        out_ref[...] = jnp.zeros_like(out_ref)
    out_ref[...] += jnp.sum(o, axis=(0, 1), keepdims=True) * (1.0 / N_NODES)


def _final(a0, a1, h, wc, bc, wo, bo, wr1, br1, wr2, br2):
    grid = N_PAD // BN
    wspec = pl.BlockSpec((DIM, DIM), lambda i: (0, 0))
    bspec = pl.BlockSpec((1, DIM), lambda i: (0, 0))
    return pl.pallas_call(
        _final_body,
        grid=(grid,),
        in_specs=[
            pl.BlockSpec((BN, HD), lambda i: (i, 0)),
            pl.BlockSpec((BN, HD), lambda i: (i, 0)),
            pl.BlockSpec((BN, DIM), lambda i: (i, 0)),
            wspec, bspec, wspec, bspec,
            wspec, bspec,
            pl.BlockSpec((DIM, 1), lambda i: (0, 0)),
            pl.BlockSpec((1, 1), lambda i: (0, 0)),
        ],
        out_specs=pl.BlockSpec((1, 1), lambda i: (0, 0)),
        out_shape=jax.ShapeDtypeStruct((1, 1), jnp.float32),
    )(a0, a1, h, wc, bc, wo, bo, wr1, br1, wr2, br2)


def kernel(node_type, edge_index, dist, emb_table, Wn, bn, We1, be1, We2, be2,
           Wc, bc, Wo, bo, Wr1, br1, Wr2, br2):
    src = jnp.pad(edge_index[0].astype(jnp.int32), (0, E_PAD - N_EDGES))
    dst = jnp.pad(edge_index[1].astype(jnp.int32), (0, E_PAD - N_EDGES),
                  constant_values=GARBAGE_ROW)
    dist_t = (jnp.pad(dist, (0, E_PAD - N_EDGES))
              .reshape(E_PAD // BE, BE // 128, 128).transpose(0, 2, 1))
    nt_p = jnp.pad(node_type.astype(jnp.int32), (0, N_PAD - N_NODES))[:, None]
    emb_pad = jnp.pad(emb_table, ((0, 128 - NTYPES), (0, 0)))

    offset = jnp.linspace(0.0, CUTOFF, NGAUSS)
    coeff = (-0.5 / ((offset[1] - offset[0]) ** 2)).reshape(1, 1)
    offs = offset.reshape(1, NGAUSS)

    e_all = _efilter3(dist_t, offs, coeff, We1, be1[:, None, :], We2,
                      be2[:, None, :])
    e_halves = [(e_all[2 * i], e_all[2 * i + 1]) for i in range(NCONV)]
    h, nh0, nh1 = _emb_nh(nt_p, emb_pad, Wn[0], bn[0][None])
    for i in range(NCONV):
        a0, a1 = _sc_conv(nh0, nh1, e_halves[i][0], e_halves[i][1], src, dst)
        if i + 1 < NCONV:
            h, nh0, nh1 = _mid(a0, a1, h, Wc[i], bc[i][None], Wo[i], bo[i][None],
                               Wn[i + 1], bn[i + 1][None])
        else:
            tot = _final(a0, a1, h, Wc[i], bc[i][None], Wo[i], bo[i][None],
                         Wr1, br1[None], Wr2, br2[None])
    return tot.reshape(1)


# speedup vs baseline: 1.0602x; 1.0379x over previous
import functools
import jax
import jax.numpy as jnp
from jax import lax
from jax.experimental import pallas as pl
from jax.experimental.pallas import tpu as pltpu
from jax.experimental.pallas import tpu_sc as plsc

N_NODES = 50000
N_EDGES = 800000
DIM = 64
HD = 32
NGAUSS = 64
NCONV = 3
CUTOFF = 5.0
NTYPES = 100
LOG2 = 0.6931471805599453

NSUB = 16
CHUNK = 80
EPT = 51200
E_PAD = EPT * NSUB
NPT = 3136
N_PAD = NPT * NSUB
GARBAGE_ROW = N_NODES

BN = 1568
BE = 2048


def _ssp(x):
    return jnp.maximum(x, 0.0) + jnp.log1p(jnp.exp(-jnp.abs(x))) - LOG2


def _dotd(a, b):
    return jnp.dot(a.astype(jnp.bfloat16), b.astype(jnp.bfloat16),
                   preferred_element_type=jnp.float32)


def _emb_nh_body(nt_ref, emb_ref, wn_ref, bn_ref, h_ref, nh0_ref, nh1_ref):
    ids = nt_ref[...]
    cols = lax.broadcasted_iota(jnp.int32, (BN, 128), 1)
    oh = (ids == cols).astype(jnp.float32)
    h = jnp.dot(oh, emb_ref[...], preferred_element_type=jnp.float32,
                precision=jax.lax.Precision.HIGHEST)
    nh = _dotd(h, wn_ref[...]) + bn_ref[...]
    h_ref[...] = h
    nh0_ref[...] = nh[:, :HD]
    nh1_ref[...] = nh[:, HD:]


def _emb_nh(nt_p, emb_pad, wn, bn):
    grid = N_PAD // BN
    return pl.pallas_call(
        _emb_nh_body,
        grid=(grid,),
        in_specs=[
            pl.BlockSpec((BN, 1), lambda i: (i, 0)),
            pl.BlockSpec((128, DIM), lambda i: (0, 0)),
            pl.BlockSpec((DIM, DIM), lambda i: (0, 0)),
            pl.BlockSpec((1, DIM), lambda i: (0, 0)),
        ],
        out_specs=[
            pl.BlockSpec((BN, DIM), lambda i: (i, 0)),
            pl.BlockSpec((BN, HD), lambda i: (i, 0)),
            pl.BlockSpec((BN, HD), lambda i: (i, 0)),
        ],
        out_shape=[
            jax.ShapeDtypeStruct((N_PAD, DIM), jnp.float32),
            jax.ShapeDtypeStruct((N_PAD, HD), jnp.float32),
            jax.ShapeDtypeStruct((N_PAD, HD), jnp.float32),
        ],
    )(nt_p, emb_pad, wn, bn)


def _efilter_body(dist_ref, offs_ref, coeff_ref, w1p_ref, b1p_ref, w2bd_ref,
                  b2p_ref, w12_ref, b12_ref, w22_ref, b22_ref, *e_refs):
    dt = dist_ref[0]
    offs = offs_ref[...]
    coeff = coeff_ref[...]
    slabs = []
    for r in range(BE // 128):
        dcol = dt[:, r:r + 1]
        d = dcol - offs
        slabs.append(jnp.exp(coeff * d * d))
    rbf = jnp.concatenate(slabs, axis=0)
    x01 = _ssp(_dotd(rbf, w1p_ref[...]) + b1p_ref[...])
    y01 = _ssp(_dotd(x01, w2bd_ref[...]) + b2p_ref[...])
    x2 = _ssp(_dotd(rbf, w12_ref[...]) + b12_ref[...])
    e2 = _ssp(_dotd(x2, w22_ref[...]) + b22_ref[...])
    e_refs[0][...] = y01[:, 0 * HD:1 * HD]
    e_refs[1][...] = y01[:, 1 * HD:2 * HD]
    e_refs[2][...] = y01[:, 2 * HD:3 * HD]
    e_refs[3][...] = y01[:, 3 * HD:4 * HD]
    e_refs[4][...] = e2[:, :HD]
    e_refs[5][...] = e2[:, HD:]


def _efilter3(dist_t, offs, coeff, w1p, b1p, w2bd, b2p, w12, b12, w22, b22):
    grid = E_PAD // BE
    return pl.pallas_call(
        _efilter_body,
        grid=(grid,),
        in_specs=[
            pl.BlockSpec((1, 128, BE // 128), lambda i: (i, 0, 0)),
            pl.BlockSpec((1, NGAUSS), lambda i: (0, 0)),
            pl.BlockSpec((1, 1), lambda i: (0, 0)),
            pl.BlockSpec((NGAUSS, 128), lambda i: (0, 0)),
            pl.BlockSpec((1, 128), lambda i: (0, 0)),
            pl.BlockSpec((128, 128), lambda i: (0, 0)),
            pl.BlockSpec((1, 128), lambda i: (0, 0)),
            pl.BlockSpec((NGAUSS, DIM), lambda i: (0, 0)),
            pl.BlockSpec((1, DIM), lambda i: (0, 0)),
            pl.BlockSpec((DIM, DIM), lambda i: (0, 0)),
            pl.BlockSpec((1, DIM), lambda i: (0, 0)),
        ],
        out_specs=[pl.BlockSpec((BE, HD), lambda i: (i, 0))] * (2 * NCONV),
        out_shape=[jax.ShapeDtypeStruct((E_PAD, HD), jnp.float32)] * (2 * NCONV),
    )(dist_t, offs, coeff, w1p, b1p, w2bd, b2p, w12, b12, w22, b22)


class _SemList:
    def __init__(self, refs):
        self.at = list(refs)


NZB = 98
NBUF = 5
NGROUP = EPT // CHUNK // NBUF


def _sc_conv_body(nh0, nh1, e0, e1, srcI, dstI, agg0, agg1,
                  src_i, dst_i, e_b, g_b, zb, aggsh, *sems):
    sem_in = _SemList(sems[0:NBUF])
    sem_g = _SemList(sems[NBUF:2 * NBUF])
    sem_s = _SemList(sems[2 * NBUF:3 * NBUF])
    cid = lax.axis_index("c")
    sid = lax.axis_index("s")

    zeros = jnp.zeros((16,), jnp.float32)

    def zrow(r, _):
        zb[r, pl.ds(0, 16)] = zeros
        zb[r, pl.ds(16, 16)] = zeros
        return 0
    lax.fori_loop(0, NZB, zrow, 0)
    for r in range(NPT // NZB):
        pltpu.sync_copy(zb, aggsh.at[pl.ds(sid * NPT + r * NZB, NZB)])
    plsc.subcore_barrier()

    def run_half(nh_h, e_h, agg_out):
        def group(go, _):
            j0 = go * NBUF

            @pl.when(go > 0)
            def _():
                for b in range(NBUF):
                    pltpu.make_async_copy(
                        g_b.at[b], aggsh.at[dst_i.at[b]], sem_s.at[b]).wait()

            for b in range(NBUF):
                base = sid * EPT + (j0 + b) * CHUNK
                pltpu.async_copy(srcI.at[pl.ds(base, CHUNK)], src_i.at[b],
                                 sem_in.at[b])
                pltpu.async_copy(dstI.at[pl.ds(base, CHUNK)], dst_i.at[b],
                                 sem_in.at[b])
                pltpu.async_copy(e_h.at[pl.ds(base, CHUNK)], e_b.at[b],
                                 sem_in.at[b])

            for b in range(NBUF):
                base = sid * EPT + (j0 + b) * CHUNK
                pltpu.make_async_copy(srcI.at[pl.ds(base, CHUNK)], src_i.at[b],
                                      sem_in.at[b]).wait()
                pltpu.make_async_copy(dstI.at[pl.ds(base, CHUNK)], dst_i.at[b],
                                      sem_in.at[b]).wait()
                pltpu.make_async_copy(e_h.at[pl.ds(base, CHUNK)], e_b.at[b],
                                      sem_in.at[b]).wait()
                pltpu.async_copy(nh_h.at[src_i.at[b]], g_b.at[b], sem_g.at[b])

            for b in range(NBUF):
                pltpu.make_async_copy(nh_h.at[src_i.at[b]], g_b.at[b],
                                      sem_g.at[b]).wait()

                def rows(i, _, b=b):
                    for rr in range(8):
                        r = i * 8 + rr
                        for c2 in range(2):
                            sl = pl.ds(c2 * 16, 16)
                            g_b[b, r, sl] = g_b[b, r, sl] * e_b[b, r, sl]
                    return 0
                lax.fori_loop(0, CHUNK // 8, rows, 0)
                pltpu.async_copy(g_b.at[b], aggsh.at[dst_i.at[b]], sem_s.at[b],
                                 add=True)
            return 0
        lax.fori_loop(0, NGROUP, group, 0)
        for b in range(NBUF):
            pltpu.make_async_copy(g_b.at[b], aggsh.at[dst_i.at[b]],
                                  sem_s.at[b]).wait()
        plsc.subcore_barrier()
        pltpu.sync_copy(aggsh.at[pl.ds(sid * NPT, NPT)],
                        agg_out.at[pl.ds(sid * NPT, NPT)])

    @pl.when(cid == 0)
    def _():
        run_half(nh0, e0, agg0)

    @pl.when(cid == 1)
    def _():
        run_half(nh1, e1, agg1)


@functools.cache
def _make_sc_conv():
    return functools.partial(
        pl.kernel,
        out_type=[
            jax.ShapeDtypeStruct((N_PAD, HD), jnp.float32),
            jax.ShapeDtypeStruct((N_PAD, HD), jnp.float32),
        ],
        mesh=plsc.VectorSubcoreMesh(core_axis_name="c", subcore_axis_name="s",
                                    num_cores=2, num_subcores=NSUB),
        scratch_types=[
            pltpu.VMEM((NBUF, CHUNK), jnp.int32),
            pltpu.VMEM((NBUF, CHUNK), jnp.int32),
            pltpu.VMEM((NBUF, CHUNK, HD), jnp.float32),
            pltpu.VMEM((NBUF, CHUNK, HD), jnp.float32),
            pltpu.VMEM((NZB, HD), jnp.float32),
            pltpu.VMEM_SHARED((N_PAD, HD), jnp.float32),
        ] + [pltpu.SemaphoreType.DMA] * (3 * NBUF) + [
        ],
        compiler_params=pltpu.CompilerParams(use_tc_tiling_on_sc=False),
    )(_sc_conv_body)


def _sc_conv(*args):
    return _make_sc_conv()(*args)


def _mid_body(a0_ref, a1_ref, h_ref, wc_ref, bc_ref, wo_ref, bo_ref,
              wn_ref, bn_ref, hn_ref, nh0_ref, nh1_ref):
    agg = jnp.concatenate([a0_ref[...], a1_ref[...]], axis=1)
    agg = _ssp(_dotd(agg, wc_ref[...]) + bc_ref[...])
    agg = _dotd(agg, wo_ref[...]) + bo_ref[...]
    hn = h_ref[...] + agg
    nh = _dotd(hn, wn_ref[...]) + bn_ref[...]
    hn_ref[...] = hn
    nh0_ref[...] = nh[:, :HD]
    nh1_ref[...] = nh[:, HD:]


def _mid(a0, a1, h, wc, bc, wo, bo, wn, bn):
    grid = N_PAD // BN
    wspec = pl.BlockSpec((DIM, DIM), lambda i: (0, 0))
    bspec = pl.BlockSpec((1, DIM), lambda i: (0, 0))
    return pl.pallas_call(
        _mid_body,
        grid=(grid,),
        in_specs=[
            pl.BlockSpec((BN, HD), lambda i: (i, 0)),
            pl.BlockSpec((BN, HD), lambda i: (i, 0)),
            pl.BlockSpec((BN, DIM), lambda i: (i, 0)),
            wspec, bspec, wspec, bspec, wspec, bspec,
        ],
        out_specs=[
            pl.BlockSpec((BN, DIM), lambda i: (i, 0)),
            pl.BlockSpec((BN, HD), lambda i: (i, 0)),
            pl.BlockSpec((BN, HD), lambda i: (i, 0)),
        ],
        out_shape=[
            jax.ShapeDtypeStruct((N_PAD, DIM), jnp.float32),
            jax.ShapeDtypeStruct((N_PAD, HD), jnp.float32),
            jax.ShapeDtypeStruct((N_PAD, HD), jnp.float32),
        ],
    )(a0, a1, h, wc, bc, wo, bo, wn, bn)


def _final_body(a0_ref, a1_ref, h_ref, wc_ref, bc_ref, wo_ref, bo_ref,
                wr1_ref, br1_ref, wr2_ref, br2_ref, out_ref):
    i = pl.program_id(0)
    agg = jnp.concatenate([a0_ref[...], a1_ref[...]], axis=1)
    agg = _ssp(_dotd(agg, wc_ref[...]) + bc_ref[...])
    agg = _dotd(agg, wo_ref[...]) + bo_ref[...]
    hn = h_ref[...] + agg
    o = _ssp(_dotd(hn, wr1_ref[...]) + br1_ref[...])
    o = _dotd(o, wr2_ref[...]) + br2_ref[...]
    rows = i * BN + lax.broadcasted_iota(jnp.int32, (BN, 1), 0)
    o = jnp.where(rows < N_NODES, o, 0.0)

    @pl.when(i == 0)
    def _():
        out_ref[...] = jnp.zeros_like(out_ref)
    out_ref[...] += jnp.sum(o, axis=(0, 1), keepdims=True) * (1.0 / N_NODES)


def _final(a0, a1, h, wc, bc, wo, bo, wr1, br1, wr2, br2):
    grid = N_PAD // BN
    wspec = pl.BlockSpec((DIM, DIM), lambda i: (0, 0))
    bspec = pl.BlockSpec((1, DIM), lambda i: (0, 0))
    return pl.pallas_call(
        _final_body,
        grid=(grid,),
        in_specs=[
            pl.BlockSpec((BN, HD), lambda i: (i, 0)),
            pl.BlockSpec((BN, HD), lambda i: (i, 0)),
            pl.BlockSpec((BN, DIM), lambda i: (i, 0)),
            wspec, bspec, wspec, bspec,
            wspec, bspec,
            pl.BlockSpec((DIM, 1), lambda i: (0, 0)),
            pl.BlockSpec((1, 1), lambda i: (0, 0)),
        ],
        out_specs=pl.BlockSpec((1, 1), lambda i: (0, 0)),
        out_shape=jax.ShapeDtypeStruct((1, 1), jnp.float32),
    )(a0, a1, h, wc, bc, wo, bo, wr1, br1, wr2, br2)


def kernel(node_type, edge_index, dist, emb_table, Wn, bn, We1, be1, We2, be2,
           Wc, bc, Wo, bo, Wr1, br1, Wr2, br2):
    src = jnp.pad(edge_index[0].astype(jnp.int32), (0, E_PAD - N_EDGES))
    dst = jnp.pad(edge_index[1].astype(jnp.int32), (0, E_PAD - N_EDGES),
                  constant_values=GARBAGE_ROW)
    dist_t = (jnp.pad(dist, (0, E_PAD - N_EDGES))
              .reshape(E_PAD // BE, BE // 128, 128).transpose(0, 2, 1))
    nt_p = jnp.pad(node_type.astype(jnp.int32), (0, N_PAD - N_NODES))[:, None]
    emb_pad = jnp.pad(emb_table, ((0, 128 - NTYPES), (0, 0)))

    offset = jnp.linspace(0.0, CUTOFF, NGAUSS)
    coeff = (-0.5 / ((offset[1] - offset[0]) ** 2)).reshape(1, 1)
    offs = offset.reshape(1, NGAUSS)

    w1p = jnp.concatenate([We1[0], We1[1]], axis=1)
    b1p = jnp.concatenate([be1[0], be1[1]])[None]
    zz = jnp.zeros((DIM, DIM), jnp.float32)
    w2bd = jnp.concatenate(
        [jnp.concatenate([We2[0], zz], axis=1),
         jnp.concatenate([zz, We2[1]], axis=1)], axis=0)
    b2p = jnp.concatenate([be2[0], be2[1]])[None]
    e_all = _efilter3(dist_t, offs, coeff, w1p, b1p, w2bd, b2p,
                      We1[2], be1[2][None], We2[2], be2[2][None])
    e_halves = [(e_all[2 * i], e_all[2 * i + 1]) for i in range(NCONV)]
    h, nh0, nh1 = _emb_nh(nt_p, emb_pad, Wn[0], bn[0][None])
    for i in range(NCONV):
        a0, a1 = _sc_conv(nh0, nh1, e_halves[i][0], e_halves[i][1], src, dst)
        if i + 1 < NCONV:
            h, nh0, nh1 = _mid(a0, a1, h, Wc[i], bc[i][None], Wo[i], bo[i][None],
                               Wn[i + 1], bn[i + 1][None])
        else:
            tot = _final(a0, a1, h, Wc[i], bc[i][None], Wo[i], bo[i][None],
                         Wr1, br1[None], Wr2, br2[None])
    return tot.reshape(1)
